# baseline (speedup 1.0000x reference)
import jax
import jax.numpy as jnp
from jax.experimental import pallas as pl
from jax.experimental.pallas import tpu as pltpu

_B = 16384
_V = 1000
_R = 512


def _bce_body(y_ref, a_ref, o_ref):
    a = a_ref[...]
    l1 = jnp.maximum(jnp.log(1.0 - a), -100.0)
    y = y_ref[...]
    col = jax.lax.broadcasted_iota(jnp.int32, (_R, _V), 1)
    t = jnp.sum(jnp.where(col == y, a, 0.0), axis=1, keepdims=True)
    lp_t = jnp.maximum(jnp.log(t), -100.0)
    l1_t = jnp.maximum(jnp.log(1.0 - t), -100.0)
    o_ref[...] = (jnp.sum(l1) + jnp.sum(lp_t - l1_t)).reshape(1, 1, 1)


def kernel(truth, prob, all_truth, y_target):
    del truth, prob
    y2 = y_target.reshape(_B, 1)
    grid = _B // _R
    partials = pl.pallas_call(
        _bce_body,
        grid=(grid,),
        in_specs=[
            pl.BlockSpec((_R, 1), lambda i: (i, 0)),
            pl.BlockSpec((_R, _V), lambda i: (i, 0)),
        ],
        out_specs=pl.BlockSpec((1, 1, 1), lambda i: (i, 0, 0)),
        out_shape=jax.ShapeDtypeStruct((grid, 1, 1), jnp.float32),
        compiler_params=pltpu.CompilerParams(
            dimension_semantics=("parallel",),
        ),
    )(y2, all_truth)
    return -jnp.sum(partials) / (_B * _V)


# speedup vs baseline: 1.1536x; 1.1536x over previous
import jax
import jax.numpy as jnp
from jax.experimental import pallas as pl
from jax.experimental.pallas import tpu as pltpu

_B = 16384
_V = 1000
_R = 2048


def _bce_body(y_ref, a_ref, o_ref):
    a = a_ref[...]
    l1 = jnp.maximum(jnp.log(1.0 - a), -100.0)
    y = y_ref[...]
    col = jax.lax.broadcasted_iota(jnp.int32, (_R, _V), 1)
    t = jnp.sum(jnp.where(col == y, a, 0.0), axis=1, keepdims=True)
    lp_t = jnp.maximum(jnp.log(t), -100.0)
    l1_t = jnp.maximum(jnp.log(1.0 - t), -100.0)
    o_ref[...] = (jnp.sum(l1) + jnp.sum(lp_t - l1_t)).reshape(1, 1, 1)


def kernel(truth, prob, all_truth, y_target):
    del truth, prob
    y2 = y_target.reshape(_B, 1)
    grid = _B // _R
    partials = pl.pallas_call(
        _bce_body,
        grid=(grid,),
        in_specs=[
            pl.BlockSpec((_R, 1), lambda i: (i, 0)),
            pl.BlockSpec((_R, _V), lambda i: (i, 0)),
        ],
        out_specs=pl.BlockSpec((1, 1, 1), lambda i: (i, 0, 0)),
        out_shape=jax.ShapeDtypeStruct((grid, 1, 1), jnp.float32),
        compiler_params=pltpu.CompilerParams(
            dimension_semantics=("parallel",),
        ),
    )(y2, all_truth)
    return -jnp.sum(partials) / (_B * _V)
